# baseline (speedup 1.0000x reference)
import functools

import jax
import jax.numpy as jnp
from jax import lax
from jax.experimental import pallas as pl
from jax.experimental.pallas import tpu as pltpu
from jax.experimental.pallas import tpu_sc as plsc

_NC = 2
_NS = 16
_NW = _NC * _NS

_IDXW = 128
_K = 8
_CH = _K * _IDXW


def _sc_gather(x2d, table, n_tok, tok_dim):
    per_w = n_tok // _NW
    n_chunks = per_w // _CH
    idx_rows_per_chunk = _K

    mesh = plsc.VectorSubcoreMesh(core_axis_name="c", subcore_axis_name="s")

    @functools.partial(
        pl.kernel,
        mesh=mesh,
        compiler_params=pltpu.CompilerParams(use_tc_tiling_on_sc=False),
        out_type=jax.ShapeDtypeStruct((n_tok, tok_dim), jnp.float32),
        scratch_types=[
            pltpu.VMEM((_K, _IDXW), jnp.int32),
            pltpu.VMEM((_CH, tok_dim), jnp.float32),
            pltpu.SemaphoreType.DMA,
        ],
    )
    def gather_kernel(x_hbm, table_hbm, emb_hbm, idx_v, rows_v, sem):
        wid = lax.axis_index("s") * _NC + lax.axis_index("c")
        row0 = wid * (per_w // _IDXW)

        def body(c, carry):
            r = row0 + c * idx_rows_per_chunk
            pltpu.sync_copy(x_hbm.at[pl.ds(r, idx_rows_per_chunk)], idx_v)
            descs = [
                pltpu.async_copy(
                    table_hbm.at[idx_v.at[j]],
                    rows_v.at[pl.ds(j * _IDXW, _IDXW)],
                    sem,
                )
                for j in range(_K)
            ]
            for d in descs:
                d.wait()
            base = r * _IDXW
            pltpu.sync_copy(rows_v, emb_hbm.at[pl.ds(base, _CH)])
            return carry

        lax.fori_loop(0, n_chunks, body, 0)

    return gather_kernel(x2d, table)


def _tc_project(xcols, emb, wt, brow, n_tok, tok_dim, emb_dim, bn):
    nb = n_tok // bn

    def body(x_ref, emb_ref, wt_ref, b_ref, out_ref):
        mask = (x_ref[0, 0, :] != 0).astype(jnp.float32)
        acc = jnp.dot(emb_ref[...], wt_ref[...],
                      preferred_element_type=jnp.float32)
        out_ref[...] = acc + mask[:, None] * b_ref[...]

    return pl.pallas_call(
        body,
        grid=(nb,),
        in_specs=[
            pl.BlockSpec((1, 1, bn), lambda i: (i, 0, 0)),
            pl.BlockSpec((bn, tok_dim), lambda i: (i, 0)),
            pl.BlockSpec((tok_dim, emb_dim), lambda i: (0, 0)),
            pl.BlockSpec((1, emb_dim), lambda i: (0, 0)),
        ],
        out_specs=pl.BlockSpec((bn, emb_dim), lambda i: (i, 0)),
        out_shape=jax.ShapeDtypeStruct((n_tok, emb_dim), jnp.float32),
    )(xcols, emb, wt, brow)


def kernel(x, table, W, b):
    bsz, seq = x.shape
    vocab, tok_dim = table.shape
    emb_dim = W.shape[0]
    n_tok = bsz * seq

    xf = x.astype(jnp.int32).reshape(n_tok)
    x2d = xf.reshape(n_tok // _IDXW, _IDXW)

    emb = _sc_gather(x2d, table, n_tok, tok_dim)

    bn = 2048
    out = _tc_project(
        xf.reshape(n_tok // bn, 1, bn), emb, W.T,
        b.reshape(1, emb_dim), n_tok, tok_dim, emb_dim, bn,
    )
    return out.reshape(bsz, seq, emb_dim)
